# baseline (speedup 1.0000x reference)
import functools

import jax
import jax.numpy as jnp
from jax.experimental import pallas as pl

PENALTY = 1.2
BV = 8192


def _damp_kernel(ids_ref, logits_ref, out_ref, *, S, V):
    vb = pl.program_id(1)
    ids = ids_ref[0]
    vids = jax.lax.broadcasted_iota(jnp.int32, (S, BV), 1) + vb * BV
    eq = ids == vids
    j = jax.lax.broadcasted_iota(jnp.int32, (S, BV), 0)
    occ = jnp.min(jnp.where(eq, j, S), axis=0, keepdims=True)
    mask = occ < j
    x = logits_ref[0]
    out_ref[0] = jnp.where(mask, x * (1.0 / PENALTY), x)


@jax.jit
def kernel(logits, input_ids):
    B, S, V = logits.shape
    nv = pl.cdiv(V, BV)
    ids3 = input_ids.reshape(B, S, 1)
    grid = (B, nv)
    return pl.pallas_call(
        functools.partial(_damp_kernel, S=S, V=V),
        grid=grid,
        in_specs=[
            pl.BlockSpec((1, S, 1), lambda b, v: (b, 0, 0)),
            pl.BlockSpec((1, S, BV), lambda b, v: (b, 0, v)),
        ],
        out_specs=pl.BlockSpec((1, S, BV), lambda b, v: (b, 0, v)),
        out_shape=jax.ShapeDtypeStruct((B, S, V), logits.dtype),
    )(ids3, logits)


# speedup vs baseline: 1.6355x; 1.6355x over previous
import jax
import jax.numpy as jnp
from jax.experimental import pallas as pl
from jax.experimental.pallas import tpu as pltpu

PENALTY = 1.2
NS = 2


def _damp_kernel(ids_ref, logits_ref, out_ref, occ_ref):
    S = ids_ref.shape[1]
    SH = logits_ref.shape[1]
    VP = occ_ref.shape[1]

    sh = pl.program_id(1)

    @pl.when(sh == 0)
    def _compute_occ():
        ids = ids_ref[0]
        vids = jax.lax.broadcasted_iota(jnp.int32, (S, VP), 1)
        j = jax.lax.broadcasted_iota(jnp.int32, (S, VP), 0)
        eq = ids == vids
        occ_ref[...] = jnp.min(jnp.where(eq, j, S), axis=0, keepdims=True)

    s_global = jax.lax.broadcasted_iota(jnp.int32, (SH, VP), 0) + sh * SH
    mask = occ_ref[...] < s_global
    x = logits_ref[0]
    out_ref[0] = jnp.where(mask, x * (1.0 / PENALTY), x)


@jax.jit
def kernel(logits, input_ids):
    B, S, V = logits.shape
    SH = S // NS
    ids3 = input_ids.reshape(B, S, 1)
    return pl.pallas_call(
        _damp_kernel,
        grid=(B, NS),
        in_specs=[
            pl.BlockSpec((1, S, 1), lambda b, s: (b, 0, 0)),
            pl.BlockSpec((1, SH, V), lambda b, s: (b, s, 0)),
        ],
        out_specs=pl.BlockSpec((1, SH, V), lambda b, s: (b, s, 0)),
        out_shape=jax.ShapeDtypeStruct((B, S, V), logits.dtype),
        scratch_shapes=[pltpu.VMEM((1, V), jnp.int32)],
    )(ids3, logits)
